# baseline (speedup 1.0000x reference)
import jax
import jax.numpy as jnp
from jax.experimental import pallas as pl
from jax.experimental.pallas import tpu as pltpu

B = 128
T = 577
VS = 768
TS = 768
D = 512


def _fused_head_kernel(vis_hbm, txt_ref, wv_ref, bv_ref, wt_ref, bt_ref,
                       out_v_ref, out_t_ref, vis_vmem, sem):
    copy = pltpu.make_async_copy(vis_hbm.at[:, 0:1, :], vis_vmem, sem)
    copy.start()
    out_t_ref[...] = (
        jnp.dot(txt_ref[...], wt_ref[...], preferred_element_type=jnp.float32)
        + bt_ref[...]
    )
    copy.wait()
    out_v_ref[...] = (
        jnp.dot(vis_vmem[:, 0, :], wv_ref[...],
                preferred_element_type=jnp.float32)
        + bv_ref[...]
    )


def kernel(visual_feature, textual_feature, attribute_feature, att_nums,
           captions, Wv, bv, Wt, bt, Wp, bp, Wa, ba):
    del attribute_feature, att_nums, captions, Wp, bp, Wa, ba
    bv2 = bv.reshape(1, D)
    bt2 = bt.reshape(1, D)
    out_v, out_t = pl.pallas_call(
        _fused_head_kernel,
        grid=(1,),
        in_specs=[
            pl.BlockSpec(memory_space=pltpu.MemorySpace.HBM),
            pl.BlockSpec((B, TS), lambda i: (0, 0)),
            pl.BlockSpec((VS, D), lambda i: (0, 0)),
            pl.BlockSpec((1, D), lambda i: (0, 0)),
            pl.BlockSpec((TS, D), lambda i: (0, 0)),
            pl.BlockSpec((1, D), lambda i: (0, 0)),
        ],
        out_specs=[
            pl.BlockSpec((B, D), lambda i: (0, 0)),
            pl.BlockSpec((B, D), lambda i: (0, 0)),
        ],
        out_shape=[
            jax.ShapeDtypeStruct((B, D), jnp.float32),
            jax.ShapeDtypeStruct((B, D), jnp.float32),
        ],
        scratch_shapes=[
            pltpu.VMEM((B, 1, VS), jnp.float32),
            pltpu.SemaphoreType.DMA,
        ],
    )(visual_feature, textual_feature, Wv, bv2, Wt, bt2)
    return (out_v, out_t)


# speedup vs baseline: 33.2365x; 33.2365x over previous
import jax
import jax.numpy as jnp
from jax.experimental import pallas as pl

B = 128
T = 577
VS = 768
TS = 768
D = 512


def _fused_head_kernel(vis_ref, txt_ref, wv_ref, bv_ref, wt_ref, bt_ref,
                       out_v_ref, out_t_ref):
    out_v_ref[...] = (
        jnp.dot(vis_ref[...], wv_ref[...],
                preferred_element_type=jnp.float32)
        + bv_ref[...]
    )
    out_t_ref[...] = (
        jnp.dot(txt_ref[...], wt_ref[...], preferred_element_type=jnp.float32)
        + bt_ref[...]
    )


def kernel(visual_feature, textual_feature, attribute_feature, att_nums,
           captions, Wv, bv, Wt, bt, Wp, bp, Wa, ba):
    del attribute_feature, att_nums, captions, Wp, bp, Wa, ba
    bv2 = bv.reshape(1, D)
    bt2 = bt.reshape(1, D)
    cls_tok = visual_feature[:, 0]
    out_v, out_t = pl.pallas_call(
        _fused_head_kernel,
        grid=(1,),
        in_specs=[
            pl.BlockSpec((B, VS), lambda i: (0, 0)),
            pl.BlockSpec((B, TS), lambda i: (0, 0)),
            pl.BlockSpec((VS, D), lambda i: (0, 0)),
            pl.BlockSpec((1, D), lambda i: (0, 0)),
            pl.BlockSpec((TS, D), lambda i: (0, 0)),
            pl.BlockSpec((1, D), lambda i: (0, 0)),
        ],
        out_specs=[
            pl.BlockSpec((B, D), lambda i: (0, 0)),
            pl.BlockSpec((B, D), lambda i: (0, 0)),
        ],
        out_shape=[
            jax.ShapeDtypeStruct((B, D), jnp.float32),
            jax.ShapeDtypeStruct((B, D), jnp.float32),
        ],
    )(cls_tok, textual_feature, Wv, bv2, Wt, bt2)
    return (out_v, out_t)


# speedup vs baseline: 49.0883x; 1.4769x over previous
import jax
import jax.numpy as jnp
from jax.experimental import pallas as pl

B = 128
T = 577
VS = 768
TS = 768
D = 512


def _fused_head_kernel(vis_ref, txt_ref, wv_ref, bv_ref, wt_ref, bt_ref,
                       out_v_ref, out_t_ref):
    out_v_ref[...] = (
        jnp.dot(vis_ref[0], wv_ref[...], preferred_element_type=jnp.float32)
        + bv_ref[...]
    )
    out_t_ref[...] = (
        jnp.dot(txt_ref[...], wt_ref[...], preferred_element_type=jnp.float32)
        + bt_ref[...]
    )


def kernel(visual_feature, textual_feature, attribute_feature, att_nums,
           captions, Wv, bv, Wt, bt, Wp, bp, Wa, ba):
    del attribute_feature, att_nums, captions, Wp, bp, Wa, ba
    bv2 = bv.reshape(1, D)
    bt2 = bt.reshape(1, D)
    vis_t = jnp.transpose(visual_feature, (1, 0, 2))
    out_v, out_t = pl.pallas_call(
        _fused_head_kernel,
        grid=(1,),
        in_specs=[
            pl.BlockSpec((1, B, VS), lambda i: (0, 0, 0)),
            pl.BlockSpec((B, TS), lambda i: (0, 0)),
            pl.BlockSpec((VS, D), lambda i: (0, 0)),
            pl.BlockSpec((1, D), lambda i: (0, 0)),
            pl.BlockSpec((TS, D), lambda i: (0, 0)),
            pl.BlockSpec((1, D), lambda i: (0, 0)),
        ],
        out_specs=[
            pl.BlockSpec((B, D), lambda i: (0, 0)),
            pl.BlockSpec((B, D), lambda i: (0, 0)),
        ],
        out_shape=[
            jax.ShapeDtypeStruct((B, D), jnp.float32),
            jax.ShapeDtypeStruct((B, D), jnp.float32),
        ],
    )(vis_t, textual_feature, Wv, bv2, Wt, bt2)
    return (out_v, out_t)
